# baseline (speedup 1.0000x reference)
import functools

import jax
import jax.numpy as jnp
from jax import lax
from jax.experimental import pallas as pl
from jax.experimental.pallas import tpu as pltpu
from jax.experimental.pallas import tpu_sc as plsc

MU_C = 1000
BATCH_C = 1024
SEQ_C = 50
M_PIECE = 40
PPS = MU_C // M_PIECE
NPIECES = SEQ_C * PPS

_INFO = plsc.get_sparse_core_info()
_NC = _INFO.num_cores
_NS = _INFO.num_subcores
_NW = _NC * _NS
_L = _INFO.num_lanes

_K_MAX = (NPIECES + _NW - 1) // _NW
_NBVEC = BATCH_C // _L
_NSROWS = 3


def _make_sc_call():
    mesh = plsc.VectorSubcoreMesh(core_axis_name="c", subcore_axis_name="s")

    @functools.partial(
        pl.kernel,
        mesh=mesh,
        out_type=jax.ShapeDtypeStruct((SEQ_C, MU_C, BATCH_C), jnp.float32),
        scratch_types=[
            pltpu.VMEM((_NSROWS * BATCH_C,), jnp.int32),
            pltpu.VMEM((M_PIECE, BATCH_C), jnp.float32),
            pltpu.VMEM((M_PIECE, BATCH_C), jnp.float32),
            pltpu.VMEM((M_PIECE, BATCH_C), jnp.float32),
            pltpu.SemaphoreType.DMA,
            pltpu.SemaphoreType.DMA,
            pltpu.SemaphoreType.DMA,
            pltpu.SemaphoreType.DMA,
        ],
        compiler_params=pltpu.CompilerParams(needs_layout_passes=False),
    )
    def one_hot_sc(
        xt_hbm, out_hbm, x_all, buf0, buf1, buf2, semx, sem0, sem1, sem2
    ):
        wid = lax.axis_index("s") * _NC + lax.axis_index("c")

        lanes = lax.iota(jnp.int32, _L)
        fzero = jnp.zeros((_L,), jnp.float32)
        fone = jnp.ones((_L,), jnp.float32)

        start = wid * NPIECES // _NW
        end = (wid + 1) * NPIECES // _NW
        s_lo = start // PPS

        def piece_params(k):
            p = start + k
            return p, p // PPS, (p % PPS) * M_PIECE

        def stage(j, carry):
            s = jnp.minimum(s_lo + j, SEQ_C - 1)
            pltpu.async_copy(
                xt_hbm.at[s], x_all.at[pl.ds(j * BATCH_C, BATCH_C)], semx
            )
            return carry

        lax.fori_loop(0, _NSROWS, stage, 0)

        def zero_row(r, carry):
            row = jnp.full((_L,), r, jnp.int32)
            for v in range(_NBVEC):
                cols = lanes + v * _L
                plsc.store_scatter(buf0, [row, cols], fzero)
                plsc.store_scatter(buf1, [row, cols], fzero)
                plsc.store_scatter(buf2, [row, cols], fzero)
            return carry

        lax.fori_loop(0, M_PIECE, zero_row, 0)

        def drain(j, carry):
            pltpu.make_async_copy(
                xt_hbm.at[0], x_all.at[pl.ds(0, BATCH_C)], semx
            ).wait()
            return carry

        lax.fori_loop(0, _NSROWS, drain, 0)

        def scan_scatter(buf, k, value):
            _, s, m0 = piece_params(k)
            srow = s - s_lo
            for v in range(_NBVEC):
                xv = x_all[pl.ds(srow * BATCH_C + v * _L, _L)]
                row = xv - m0
                m = plsc.bitcast(row, jnp.uint32) < M_PIECE
                cols = lanes + v * _L
                plsc.store_scatter(buf, [row, cols], value, mask=m)

        def _wait_piece(buf, sem):
            pltpu.make_async_copy(
                buf, out_hbm.at[0, pl.ds(0, M_PIECE)], sem
            ).wait()

        def do_piece(k3, k, buf, sem):
            _, s, m0 = piece_params(k)

            @pl.when(k3 > 0)
            def _():
                _wait_piece(buf, sem)
                scan_scatter(buf, k - 3, fzero)

            scan_scatter(buf, k, fone)
            pltpu.async_copy(buf, out_hbm.at[s, pl.ds(m0, M_PIECE)], sem)

        bufs = (buf0, buf1, buf2)
        sems = (sem0, sem1, sem2)

        def main_body(k3, carry):
            for j in range(3):
                k = 3 * k3 + j

                @pl.when(start + k < end)
                def _(k3=k3, k=k, j=j):
                    do_piece(k3, k, bufs[j], sems[j])

            return carry

        lax.fori_loop(0, (_K_MAX + 2) // 3, main_body, 0)

        _wait_piece(buf0, sem0)
        _wait_piece(buf1, sem1)
        _wait_piece(buf2, sem2)

    return one_hot_sc


_sc_call = _make_sc_call()


@jax.jit
def kernel(x, ones):
    del ones
    xt = jnp.transpose(x.astype(jnp.int32), (1, 0))
    out_smb = _sc_call(xt)
    return jnp.transpose(out_smb, (2, 1, 0))


# speedup vs baseline: 1.0212x; 1.0212x over previous
import functools

import jax
import jax.numpy as jnp
from jax import lax
from jax.experimental import pallas as pl
from jax.experimental.pallas import tpu as pltpu
from jax.experimental.pallas import tpu_sc as plsc

MU_C = 1000
BATCH_C = 1024
SEQ_C = 50
M_PIECE = 40
PPS = MU_C // M_PIECE
NPIECES = SEQ_C * PPS

_INFO = plsc.get_sparse_core_info()
_NC = _INFO.num_cores
_NS = _INFO.num_subcores
_NW = _NC * _NS
_L = _INFO.num_lanes

_K_MAX = (NPIECES + _NW - 1) // _NW
_NBVEC = BATCH_C // _L
_NSROWS = 3


def _make_sc_call():
    mesh = plsc.VectorSubcoreMesh(core_axis_name="c", subcore_axis_name="s")

    @functools.partial(
        pl.kernel,
        mesh=mesh,
        out_type=jax.ShapeDtypeStruct((SEQ_C, MU_C, BATCH_C), jnp.float32),
        scratch_types=[
            pltpu.VMEM((_NSROWS * BATCH_C,), jnp.int32),
            pltpu.VMEM((M_PIECE, BATCH_C), jnp.float32),
            pltpu.VMEM((M_PIECE, BATCH_C), jnp.float32),
            pltpu.SemaphoreType.DMA,
            pltpu.SemaphoreType.DMA,
            pltpu.SemaphoreType.DMA,
        ],
        compiler_params=pltpu.CompilerParams(needs_layout_passes=False),
    )
    def one_hot_sc(xt_hbm, out_hbm, x_all, buf0, buf1, semx, sem0, sem1):
        wid = lax.axis_index("s") * _NC + lax.axis_index("c")

        lanes = lax.iota(jnp.int32, _L)
        fzero = jnp.zeros((_L,), jnp.float32)
        fone = jnp.ones((_L,), jnp.float32)

        start = wid * NPIECES // _NW
        end = (wid + 1) * NPIECES // _NW
        s_lo = start // PPS

        def piece_params(k):
            p = start + k
            return p, p // PPS, (p % PPS) * M_PIECE

        def stage(j, carry):
            s = jnp.minimum(s_lo + j, SEQ_C - 1)
            pltpu.async_copy(
                xt_hbm.at[s], x_all.at[pl.ds(j * BATCH_C, BATCH_C)], semx
            )
            return carry

        lax.fori_loop(0, _NSROWS, stage, 0)

        def zero_row(r, carry):
            row = jnp.full((_L,), r, jnp.int32)
            for v in range(_NBVEC):
                cols = lanes + v * _L
                plsc.store_scatter(buf0, [row, cols], fzero)
                plsc.store_scatter(buf1, [row, cols], fzero)
            return carry

        lax.fori_loop(0, M_PIECE, zero_row, 0)

        def drain(j, carry):
            pltpu.make_async_copy(
                xt_hbm.at[0], x_all.at[pl.ds(0, BATCH_C)], semx
            ).wait()
            return carry

        lax.fori_loop(0, _NSROWS, drain, 0)

        def scan_scatter(buf, k, value):
            _, s, m0 = piece_params(k)
            srow = s - s_lo
            for v in range(_NBVEC):
                xv = x_all[pl.ds(srow * BATCH_C + v * _L, _L)]
                row = xv - m0
                m = plsc.bitcast(row, jnp.uint32) < M_PIECE
                cols = lanes + v * _L
                plsc.store_scatter(buf, [row, cols], value, mask=m)

        def _wait_piece(buf, sem):
            pltpu.make_async_copy(
                buf, out_hbm.at[0, pl.ds(0, M_PIECE)], sem
            ).wait()

        def do_piece(k2, k, buf, sem):
            p, s, m0 = piece_params(k)

            @pl.when(k2 > 0)
            def _():
                _wait_piece(buf, sem)
                scan_scatter(buf, k - 2, fzero)

            scan_scatter(buf, k, fone)
            pltpu.async_copy(buf, out_hbm.at[s, pl.ds(m0, M_PIECE)], sem)

        def main_body(k2, carry):
            do_piece(k2, 2 * k2, buf0, sem0)

            @pl.when(start + 2 * k2 + 1 < end)
            def _():
                do_piece(k2, 2 * k2 + 1, buf1, sem1)

            return carry

        lax.fori_loop(0, _K_MAX // 2, main_body, 0)

        _wait_piece(buf0, sem0)
        _wait_piece(buf1, sem1)

    return one_hot_sc


_sc_call = _make_sc_call()


@jax.jit
def kernel(x, ones):
    del ones
    xt = jnp.transpose(x.astype(jnp.int32), (1, 0))
    out_smb = _sc_call(xt)
    return jnp.transpose(out_smb, (2, 1, 0))
